# Initial kernel scaffold; baseline (speedup 1.0000x reference)
import jax
import jax.numpy as jnp
from jax.experimental import pallas as pl


def kernel(x, idx, W, b):
    raise NotImplementedError("write your pallas kernel here")


# speedup vs baseline: 2.6175x; 2.6175x over previous
import functools

import jax
import jax.numpy as jnp
from jax import lax
from jax.experimental import pallas as pl
from jax.experimental.pallas import tpu as pltpu
from jax.experimental.pallas import tpu_sc as plsc

N = 320000
D = 128
S = 10000
DO = 256

NW = 32
SEG_BIG = 320
SEG_SMALL = 312
TRASH = SEG_BIG
ACC_ROWS = SEG_BIG + 1
T = 256
NEG_INF = float("-inf")


def _seg_start(w):
    return SEG_SMALL * w + 8 * jnp.minimum(w, 2) if not isinstance(w, int) \
        else SEG_SMALL * w + 8 * min(w, 2)


def _seg_max_body(x_hbm, idx_hbm, q_hbm, xg_hbm, acc, xbuf, ibuf, qbuf):
    wid = lax.axis_index("c") * 16 + lax.axis_index("s")
    s0 = pl.multiple_of(_seg_start(wid), 8)
    nseg = jnp.where(wid < 2, SEG_BIG, SEG_SMALL)

    neg = jnp.full((16,), NEG_INF, dtype=jnp.float32)

    def init_row(r, _):
        for j in range(8):
            acc[r, pl.ds(16 * j, 16)] = neg
        return 0

    lax.fori_loop(0, ACC_ROWS, init_row, 0)

    pltpu.sync_copy(q_hbm, qbuf)
    qv = qbuf[pl.ds(wid, 16)]
    q0 = qv[0]
    q1 = qv[1]
    a0 = (q0 // 8) * 8
    a1 = ((q1 + 7) // 8) * 8
    n = a1 - a0
    ntiles = (n + T - 1) // T

    def point_body(i, carry):
        cur_rel = carry[0]
        regs = carry[1:]
        seg = ibuf[pl.ds(i, 16)][0]
        rel = seg - s0
        valid = jnp.logical_and(rel >= 0, rel < nseg)
        rel = jnp.where(valid, rel, TRASH)
        changed = rel != cur_rel

        @pl.when(changed)
        def _flush():
            for j in range(8):
                sl = pl.ds(16 * j, 16)
                acc[cur_rel, sl] = jnp.maximum(acc[cur_rel, sl], regs[j])

        new_regs = []
        for j in range(8):
            row = xbuf[i, pl.ds(16 * j, 16)]
            new_regs.append(jnp.where(changed, row, jnp.maximum(regs[j], row)))
        return (rel,) + tuple(new_regs)

    def tile_body(t, carry):
        start = a0 + t * T
        start = pl.multiple_of(jnp.clip(start, 0, N - T), 8)
        pltpu.sync_copy(x_hbm.at[pl.ds(start, T)], xbuf)
        pltpu.sync_copy(idx_hbm.at[pl.ds(start, T)], ibuf.at[pl.ds(0, T)])
        return lax.fori_loop(0, T, point_body, carry)

    carry0 = (jnp.int32(TRASH),) + tuple(neg for _ in range(8))
    carry = lax.fori_loop(0, ntiles, tile_body, carry0)

    cur_rel = carry[0]
    for j in range(8):
        sl = pl.ds(16 * j, 16)
        acc[cur_rel, sl] = jnp.maximum(acc[cur_rel, sl], carry[1 + j])

    def fix_row(r, _):
        for j in range(8):
            sl = pl.ds(16 * j, 16)
            v = acc[r, sl]
            acc[r, sl] = jnp.where(v == NEG_INF, 0.0, v)
        return 0

    lax.fori_loop(0, ACC_ROWS, fix_row, 0)

    @pl.when(wid < 2)
    def _out_big():
        pltpu.sync_copy(acc.at[pl.ds(0, SEG_BIG)],
                        xg_hbm.at[pl.ds(s0, SEG_BIG)])

    @pl.when(wid >= 2)
    def _out_small():
        pltpu.sync_copy(acc.at[pl.ds(0, SEG_SMALL)],
                        xg_hbm.at[pl.ds(s0, SEG_SMALL)])


def _segment_max_sc(x, idx, q):
    mesh = plsc.VectorSubcoreMesh(core_axis_name="c", subcore_axis_name="s")
    f = functools.partial(
        pl.kernel,
        out_type=jax.ShapeDtypeStruct((S, D), jnp.float32),
        mesh=mesh,
        scratch_types=[
            pltpu.VMEM((ACC_ROWS, D), jnp.float32),
            pltpu.VMEM((T, D), jnp.float32),
            pltpu.VMEM((T + 16,), jnp.int32),
            pltpu.VMEM((48,), jnp.int32),
        ],
    )(_seg_max_body)
    return f(x, idx, q)


def _mlp_body(xg_ref, w_ref, b_ref, o_ref):
    y = jnp.dot(xg_ref[...], w_ref[...], preferred_element_type=jnp.float32)
    o_ref[...] = jnp.maximum(y + b_ref[...], 0.0)


def _mlp_tc(xg, W, b):
    return pl.pallas_call(
        _mlp_body,
        grid=(10,),
        in_specs=[
            pl.BlockSpec((S // 10, D), lambda i: (i, 0)),
            pl.BlockSpec((D, DO), lambda i: (0, 0)),
            pl.BlockSpec((1, DO), lambda i: (0, 0)),
        ],
        out_specs=pl.BlockSpec((S // 10, DO), lambda i: (i, 0)),
        out_shape=jax.ShapeDtypeStruct((S, DO), jnp.float32),
    )(xg, W, b.reshape(1, DO))


def kernel(x, idx, W, b):
    w_arange = jnp.arange(NW, dtype=jnp.int32)
    seg_starts = SEG_SMALL * w_arange + 8 * jnp.minimum(w_arange, 2)
    q = jnp.searchsorted(idx, seg_starts, side="left").astype(jnp.int32)
    q = jnp.concatenate([q, jnp.full((16,), N, dtype=jnp.int32)])
    xg = _segment_max_sc(x, idx, q)
    return _mlp_tc(xg, W, b)


# speedup vs baseline: 5.2404x; 2.0021x over previous
import functools

import jax
import jax.numpy as jnp
from jax import lax
from jax.experimental import pallas as pl
from jax.experimental.pallas import tpu as pltpu
from jax.experimental.pallas import tpu_sc as plsc

N = 320000
D = 128
S = 10000
DO = 256

NW = 32
SEG_BIG = 320
SEG_SMALL = 312
TRASH = SEG_BIG
ACC_ROWS = SEG_BIG + 1
T = 256
NEG_INF = float("-inf")


def _seg_start(w):
    return SEG_SMALL * w + 8 * jnp.minimum(w, 2) if not isinstance(w, int) \
        else SEG_SMALL * w + 8 * min(w, 2)


def _seg_max_body(x_hbm, idx_hbm, q_hbm, xg_hbm,
                  acc, xbuf0, xbuf1, ibuf0, ibuf1, qbuf,
                  semx0, semx1, semi0, semi1):
    wid = lax.axis_index("c") * 16 + lax.axis_index("s")
    s0 = pl.multiple_of(_seg_start(wid), 8)
    nseg = jnp.where(wid < 2, SEG_BIG, SEG_SMALL)

    neg = jnp.full((16,), NEG_INF, dtype=jnp.float32)

    def init_row(r, _):
        for j in range(8):
            acc[r, pl.ds(16 * j, 16)] = neg
        return 0

    lax.fori_loop(0, ACC_ROWS, init_row, 0)

    pltpu.sync_copy(q_hbm, qbuf)
    qv = qbuf[pl.ds(wid, 16)]
    q0 = qv[0]
    q1 = qv[1]
    a0 = (q0 // 8) * 8
    a1 = ((q1 + 7) // 8) * 8
    n = a1 - a0
    ntiles = (n + T - 1) // T

    def tile_start(t):
        return pl.multiple_of(jnp.clip(a0 + t * T, 0, N - T), 8)

    def start_fetch(t, xb, ib, sx, si):
        st = tile_start(t)
        pltpu.async_copy(x_hbm.at[pl.ds(st, T)], xb, sx)
        pltpu.async_copy(idx_hbm.at[pl.ds(st, T)], ib.at[pl.ds(0, T)], si)

    def make_proc(xb, ib, sx, si):
        def proc():
            pltpu.make_async_copy(x_hbm.at[pl.ds(0, T)], xb, sx).wait()
            pltpu.make_async_copy(idx_hbm.at[pl.ds(0, T)], ib.at[pl.ds(0, T)],
                                  si).wait()

            def group_body(g, carry):
                idxv = ib[pl.ds(16 * g, 16)]
                cur_rel = carry[0]
                regs = list(carry[1:])
                for k in range(16):
                    i = 16 * g + k
                    seg = idxv[k]
                    rel = seg - s0
                    valid = jnp.logical_and(rel >= 0, rel < nseg)
                    rel = jnp.where(valid, rel, TRASH)
                    changed = rel != cur_rel

                    @pl.when(changed)
                    def _flush(cur_rel=cur_rel, regs=tuple(regs)):
                        for j in range(8):
                            sl = pl.ds(16 * j, 16)
                            acc[cur_rel, sl] = jnp.maximum(acc[cur_rel, sl],
                                                           regs[j])

                    for j in range(8):
                        row = xb[i, pl.ds(16 * j, 16)]
                        regs[j] = jnp.where(changed, row,
                                            jnp.maximum(regs[j], row))
                    cur_rel = rel
                return (cur_rel,) + tuple(regs)

            carry0 = (jnp.int32(TRASH),) + tuple(neg for _ in range(8))
            carry = lax.fori_loop(0, T // 16, group_body, carry0)
            cur_rel = carry[0]
            for j in range(8):
                sl = pl.ds(16 * j, 16)
                acc[cur_rel, sl] = jnp.maximum(acc[cur_rel, sl], carry[1 + j])
        return proc

    proc0 = make_proc(xbuf0, ibuf0, semx0, semi0)
    proc1 = make_proc(xbuf1, ibuf1, semx1, semi1)

    @pl.when(ntiles > 0)
    def _prologue0():
        start_fetch(0, xbuf0, ibuf0, semx0, semi0)

    @pl.when(ntiles > 1)
    def _prologue1():
        start_fetch(1, xbuf1, ibuf1, semx1, semi1)

    def tile_body(t, _):
        even = (t & 1) == 0

        @pl.when(even)
        def _do0():
            proc0()

            @pl.when(t + 2 < ntiles)
            def _pf0():
                start_fetch(t + 2, xbuf0, ibuf0, semx0, semi0)

        @pl.when(jnp.logical_not(even))
        def _do1():
            proc1()

            @pl.when(t + 2 < ntiles)
            def _pf1():
                start_fetch(t + 2, xbuf1, ibuf1, semx1, semi1)

        return 0

    lax.fori_loop(0, ntiles, tile_body, 0)

    def fix_row(r, _):
        for j in range(8):
            sl = pl.ds(16 * j, 16)
            v = acc[r, sl]
            acc[r, sl] = jnp.where(v == NEG_INF, 0.0, v)
        return 0

    lax.fori_loop(0, ACC_ROWS, fix_row, 0)

    @pl.when(wid < 2)
    def _out_big():
        pltpu.sync_copy(acc.at[pl.ds(0, SEG_BIG)],
                        xg_hbm.at[pl.ds(s0, SEG_BIG)])

    @pl.when(wid >= 2)
    def _out_small():
        pltpu.sync_copy(acc.at[pl.ds(0, SEG_SMALL)],
                        xg_hbm.at[pl.ds(s0, SEG_SMALL)])


def _segment_max_sc(x, idx, q):
    mesh = plsc.VectorSubcoreMesh(core_axis_name="c", subcore_axis_name="s")
    f = functools.partial(
        pl.kernel,
        out_type=jax.ShapeDtypeStruct((S, D), jnp.float32),
        mesh=mesh,
        scratch_types=[
            pltpu.VMEM((ACC_ROWS, D), jnp.float32),
            pltpu.VMEM((T, D), jnp.float32),
            pltpu.VMEM((T, D), jnp.float32),
            pltpu.VMEM((T + 16,), jnp.int32),
            pltpu.VMEM((T + 16,), jnp.int32),
            pltpu.VMEM((48,), jnp.int32),
            pltpu.SemaphoreType.DMA,
            pltpu.SemaphoreType.DMA,
            pltpu.SemaphoreType.DMA,
            pltpu.SemaphoreType.DMA,
        ],
    )(_seg_max_body)
    return f(x, idx, q)


def _mlp_body(xg_ref, w_ref, b_ref, o_ref):
    y = jnp.dot(xg_ref[...], w_ref[...], preferred_element_type=jnp.float32)
    o_ref[...] = jnp.maximum(y + b_ref[...], 0.0)


def _mlp_tc(xg, W, b):
    return pl.pallas_call(
        _mlp_body,
        grid=(10,),
        in_specs=[
            pl.BlockSpec((S // 10, D), lambda i: (i, 0)),
            pl.BlockSpec((D, DO), lambda i: (0, 0)),
            pl.BlockSpec((1, DO), lambda i: (0, 0)),
        ],
        out_specs=pl.BlockSpec((S // 10, DO), lambda i: (i, 0)),
        out_shape=jax.ShapeDtypeStruct((S, DO), jnp.float32),
    )(xg, W, b.reshape(1, DO))


def kernel(x, idx, W, b):
    w_arange = jnp.arange(NW, dtype=jnp.int32)
    seg_starts = SEG_SMALL * w_arange + 8 * jnp.minimum(w_arange, 2)
    q = jnp.searchsorted(idx, seg_starts, side="left").astype(jnp.int32)
    q = jnp.concatenate([q, jnp.full((16,), N, dtype=jnp.int32)])
    xg = _segment_max_sc(x, idx, q)
    return _mlp_tc(xg, W, b)


# speedup vs baseline: 7.5470x; 1.4402x over previous
import functools

import jax
import jax.numpy as jnp
from jax import lax
from jax.experimental import pallas as pl
from jax.experimental.pallas import tpu as pltpu
from jax.experimental.pallas import tpu_sc as plsc

N = 320000
D = 128
S = 10000
DO = 256

NW = 32
SEG_BIG = 320
SEG_SMALL = 312
TRASH = SEG_BIG
CARRY_ROW = SEG_BIG + 1
ACC_ROWS = SEG_BIG + 2
T = 256
NEG_INF = float("-inf")


def _bcast_lane(v, k):
    idxs = jnp.full((16, 1), k, dtype=jnp.int32)
    dn = lax.GatherDimensionNumbers(offset_dims=(), collapsed_slice_dims=(0,),
                                    start_index_map=(0,))
    return lax.gather(v, idxs, dn, (1,),
                      mode=lax.GatherScatterMode.PROMISE_IN_BOUNDS)


def _seg_start(w):
    return SEG_SMALL * w + 8 * jnp.minimum(w, 2) if not isinstance(w, int) \
        else SEG_SMALL * w + 8 * min(w, 2)


def _seg_max_body(x_hbm, idx_hbm, q_hbm, xg_hbm,
                  acc, xbuf0, xbuf1, ibuf0, ibuf1, qbuf, rbuf,
                  semx0, semx1, semi0, semi1):
    wid = lax.axis_index("c") * 16 + lax.axis_index("s")
    s0 = pl.multiple_of(_seg_start(wid), 8)
    nseg = jnp.where(wid < 2, SEG_BIG, SEG_SMALL)

    neg = jnp.full((16,), NEG_INF, dtype=jnp.float32)
    zero = jnp.zeros((16,), dtype=jnp.float32)

    def init_row(r, _):
        acc[pl.ds(r * 16, 16)] = zero
        return 0

    lax.fori_loop(0, ACC_ROWS * 8, init_row, 0)

    pltpu.sync_copy(q_hbm, qbuf)
    qv = qbuf[pl.ds(wid, 16)]
    q0 = qv[0]
    q1 = qv[1]
    a0 = (q0 // 16) * 16
    ntiles = (q1 - a0 + T - 1) // T

    lanes = lax.iota(jnp.int32, 16)
    rbuf[...] = lanes + TRASH * 128
    for j in range(8):
        acc[pl.ds(CARRY_ROW * 128 + 16 * j, 16)] = neg

    def tile_start(t):
        return pl.multiple_of(jnp.minimum(a0 + t * T, N - T), 8)

    def start_fetch(t, xb, ib, sx, si):
        st = tile_start(t)
        pltpu.async_copy(x_hbm.at[pl.ds(st, T)], xb, sx)
        pltpu.async_copy(idx_hbm.at[pl.ds(st, T)], ib.at[pl.ds(0, T)], si)

    def make_proc(xb, ib, sx, si):
        def proc(t):
            pltpu.make_async_copy(x_hbm.at[pl.ds(0, T)], xb, sx).wait()
            pltpu.make_async_copy(idx_hbm.at[pl.ds(0, T)], ib.at[pl.ds(0, T)],
                                  si).wait()
            goff = jnp.maximum(a0 + t * T - (N - T), 0) // 16

            cur0 = rbuf[...]
            regs0 = tuple(acc[pl.ds(CARRY_ROW * 128 + 16 * j, 16)]
                          for j in range(8))

            def group_body(g, carry):
                idxv = ib[pl.ds(16 * g, 16)]
                relv = idxv - s0
                validv = jnp.logical_and(relv >= 0, relv < nseg)
                woffv = jnp.where(validv, relv, TRASH) << 7
                cur_av = carry[0]
                regs = list(carry[1:])
                for k in range(16):
                    i = 16 * g + k
                    av = _bcast_lane(woffv, k) + lanes
                    changed_m = av != cur_av
                    rows = [None] * 8
                    for j in range(8):
                        rows[j] = xb[i, pl.ds(16 * j, 16)]
                        plsc.store_scatter(
                            acc.at[pl.ds(16 * j, (ACC_ROWS - 1) * 128 + 16)],
                            [cur_av], regs[j], mask=changed_m)
                    for j in range(8):
                        regs[j] = jnp.where(changed_m, rows[j],
                                            jnp.maximum(regs[j], rows[j]))
                    cur_av = av
                return (cur_av,) + tuple(regs)

            carry = plsc.parallel_loop(
                goff, T // 16, carry=(cur0,) + regs0)(group_body)
            rbuf[...] = carry[0]
            for j in range(8):
                acc[pl.ds(CARRY_ROW * 128 + 16 * j, 16)] = carry[1 + j]
        return proc

    proc0 = make_proc(xbuf0, ibuf0, semx0, semi0)
    proc1 = make_proc(xbuf1, ibuf1, semx1, semi1)

    @pl.when(ntiles > 0)
    def _prologue0():
        start_fetch(0, xbuf0, ibuf0, semx0, semi0)

    @pl.when(ntiles > 1)
    def _prologue1():
        start_fetch(1, xbuf1, ibuf1, semx1, semi1)

    def tile_body(t, _):
        even = (t & 1) == 0

        @pl.when(even)
        def _do0():
            proc0(t)

            @pl.when(t + 2 < ntiles)
            def _pf0():
                start_fetch(t + 2, xbuf0, ibuf0, semx0, semi0)

        @pl.when(jnp.logical_not(even))
        def _do1():
            proc1(t)

            @pl.when(t + 2 < ntiles)
            def _pf1():
                start_fetch(t + 2, xbuf1, ibuf1, semx1, semi1)

        return 0

    lax.fori_loop(0, ntiles, tile_body, 0)

    last_av = rbuf[...]
    for j in range(8):
        plsc.store_scatter(acc, [last_av + 16 * j],
                           acc[pl.ds(CARRY_ROW * 128 + 16 * j, 16)])

    s0w = pl.multiple_of(s0 * 128, 8)

    @pl.when(wid < 2)
    def _out_big():
        pltpu.sync_copy(acc.at[pl.ds(0, SEG_BIG * 128)],
                        xg_hbm.at[pl.ds(s0w, SEG_BIG * 128)])

    @pl.when(wid >= 2)
    def _out_small():
        pltpu.sync_copy(acc.at[pl.ds(0, SEG_SMALL * 128)],
                        xg_hbm.at[pl.ds(s0w, SEG_SMALL * 128)])


def _segment_max_sc(x, idx, q):
    mesh = plsc.VectorSubcoreMesh(core_axis_name="c", subcore_axis_name="s")
    f = functools.partial(
        pl.kernel,
        out_type=jax.ShapeDtypeStruct((S * D,), jnp.float32),
        mesh=mesh,
        scratch_types=[
            pltpu.VMEM((ACC_ROWS * D,), jnp.float32),
            pltpu.VMEM((T, D), jnp.float32),
            pltpu.VMEM((T, D), jnp.float32),
            pltpu.VMEM((T + 16,), jnp.int32),
            pltpu.VMEM((T + 16,), jnp.int32),
            pltpu.VMEM((48,), jnp.int32),
            pltpu.VMEM((16,), jnp.int32),
            pltpu.SemaphoreType.DMA,
            pltpu.SemaphoreType.DMA,
            pltpu.SemaphoreType.DMA,
            pltpu.SemaphoreType.DMA,
        ],
        compiler_params=pltpu.CompilerParams(needs_layout_passes=False),
    )(_seg_max_body)
    return f(x, idx, q).reshape(S, D)


def _mlp_body(xg_ref, w_ref, b_ref, o_ref):
    y = jnp.dot(xg_ref[...], w_ref[...], preferred_element_type=jnp.float32)
    o_ref[...] = jnp.maximum(y + b_ref[...], 0.0)


def _mlp_tc(xg, W, b):
    return pl.pallas_call(
        _mlp_body,
        grid=(10,),
        in_specs=[
            pl.BlockSpec((S // 10, D), lambda i: (i, 0)),
            pl.BlockSpec((D, DO), lambda i: (0, 0)),
            pl.BlockSpec((1, DO), lambda i: (0, 0)),
        ],
        out_specs=pl.BlockSpec((S // 10, DO), lambda i: (i, 0)),
        out_shape=jax.ShapeDtypeStruct((S, DO), jnp.float32),
    )(xg, W, b.reshape(1, DO))


def kernel(x, idx, W, b):
    w_arange = jnp.arange(NW, dtype=jnp.int32)
    seg_starts = SEG_SMALL * w_arange + 8 * jnp.minimum(w_arange, 2)
    q = jnp.searchsorted(idx, seg_starts, side="left").astype(jnp.int32)
    q = jnp.concatenate([q, jnp.full((16,), N, dtype=jnp.int32)])
    xg = _segment_max_sc(x, idx, q)
    return _mlp_tc(xg, W, b)


# speedup vs baseline: 9.7612x; 1.2934x over previous
import functools

import jax
import jax.numpy as jnp
from jax import lax
from jax.experimental import pallas as pl
from jax.experimental.pallas import tpu as pltpu
from jax.experimental.pallas import tpu_sc as plsc

N = 320000
D = 128
S = 10000
DO = 256

NW = 32
SEG_BIG = 320
SEG_SMALL = 312
TRASH = SEG_BIG
CARRY_ROW = SEG_BIG + 1
ACC_ROWS = SEG_BIG + 2
T = 256
NEG_INF = float("-inf")


def _bcast_lane(v, k):
    idxs = jnp.full((16, 1), k, dtype=jnp.int32)
    dn = lax.GatherDimensionNumbers(offset_dims=(), collapsed_slice_dims=(0,),
                                    start_index_map=(0,))
    return lax.gather(v, idxs, dn, (1,),
                      mode=lax.GatherScatterMode.PROMISE_IN_BOUNDS)


def _seg_start(w):
    return SEG_SMALL * w + 8 * jnp.minimum(w, 2) if not isinstance(w, int) \
        else SEG_SMALL * w + 8 * min(w, 2)


def _seg_max_body(x_hbm, idx_hbm, q_hbm, xg_hbm,
                  acc, xbuf0, xbuf1, ibuf0, ibuf1, qbuf, rbuf,
                  semx0, semx1, semi0, semi1):
    wid = lax.axis_index("c") * 16 + lax.axis_index("s")
    s0 = pl.multiple_of(_seg_start(wid), 8)
    nseg = jnp.where(wid < 2, SEG_BIG, SEG_SMALL)

    neg = jnp.full((16,), NEG_INF, dtype=jnp.float32)
    zero = jnp.zeros((16,), dtype=jnp.float32)

    def init_row(r, _):
        for j in range(8):
            acc[pl.ds(r * 128 + 16 * j, 16)] = zero
        return 0

    lax.fori_loop(0, ACC_ROWS, init_row, 0)

    pltpu.sync_copy(q_hbm, qbuf)
    qv = qbuf[pl.ds(wid, 16)]
    q0 = qv[0]
    q1 = qv[1]
    a0 = (q0 // 16) * 16
    ntiles = (q1 - a0 + T - 1) // T

    lanes = lax.iota(jnp.int32, 16)
    rbuf[...] = lanes + TRASH * 128
    for j in range(8):
        acc[pl.ds(CARRY_ROW * 128 + 16 * j, 16)] = neg

    def tile_start(t):
        return pl.multiple_of(jnp.minimum(a0 + t * T, N - T), 8)

    def start_fetch(t, xb, ib, sx, si):
        st = tile_start(t)
        pltpu.async_copy(x_hbm.at[pl.ds(st, T)], xb, sx)
        pltpu.async_copy(idx_hbm.at[pl.ds(st, T)], ib.at[pl.ds(0, T)], si)

    def make_proc(xb, ib, sx, si):
        def proc(t):
            pltpu.make_async_copy(x_hbm.at[pl.ds(0, T)], xb, sx).wait()
            pltpu.make_async_copy(idx_hbm.at[pl.ds(0, T)], ib.at[pl.ds(0, T)],
                                  si).wait()
            goff = jnp.maximum(a0 + t * T - (N - T), 0) // 16

            cur0 = rbuf[...]
            regs0 = tuple(acc[pl.ds(CARRY_ROW * 128 + 16 * j, 16)]
                          for j in range(8))

            def head(g):
                idxv = ib[pl.ds(16 * g, 16)]
                relv = idxv - s0
                validv = jnp.logical_and(relv >= 0, relv < nseg)
                return jnp.where(validv, relv, TRASH) << 7

            def group_body(g, carry):
                woffv = carry[0]
                woffv_nxt = head(g + 1)
                cur_av = carry[1]
                regs = list(carry[2:])
                for k in range(16):
                    i = 16 * g + k
                    av = _bcast_lane(woffv, k) + lanes
                    changed_m = av != cur_av
                    rows = [None] * 8
                    for j in range(8):
                        rows[j] = xb[i, pl.ds(16 * j, 16)]
                        plsc.store_scatter(
                            acc.at[pl.ds(16 * j, (ACC_ROWS - 1) * 128 + 16)],
                            [cur_av], regs[j], mask=changed_m)
                    for j in range(8):
                        regs[j] = jnp.where(changed_m, rows[j],
                                            jnp.maximum(regs[j], rows[j]))
                    cur_av = av
                return (woffv_nxt, cur_av) + tuple(regs)

            carry = lax.fori_loop(goff, T // 16, group_body,
                                  (head(goff), cur0) + regs0)
            rbuf[...] = carry[1]
            for j in range(8):
                acc[pl.ds(CARRY_ROW * 128 + 16 * j, 16)] = carry[2 + j]
        return proc

    proc0 = make_proc(xbuf0, ibuf0, semx0, semi0)
    proc1 = make_proc(xbuf1, ibuf1, semx1, semi1)

    @pl.when(ntiles > 0)
    def _prologue0():
        start_fetch(0, xbuf0, ibuf0, semx0, semi0)

    @pl.when(ntiles > 1)
    def _prologue1():
        start_fetch(1, xbuf1, ibuf1, semx1, semi1)

    def tile_body(t, _):
        even = (t & 1) == 0

        @pl.when(even)
        def _do0():
            proc0(t)

            @pl.when(t + 2 < ntiles)
            def _pf0():
                start_fetch(t + 2, xbuf0, ibuf0, semx0, semi0)

        @pl.when(jnp.logical_not(even))
        def _do1():
            proc1(t)

            @pl.when(t + 2 < ntiles)
            def _pf1():
                start_fetch(t + 2, xbuf1, ibuf1, semx1, semi1)

        return 0

    lax.fori_loop(0, ntiles, tile_body, 0)

    last_av = rbuf[...]
    for j in range(8):
        plsc.store_scatter(acc, [last_av + 16 * j],
                           acc[pl.ds(CARRY_ROW * 128 + 16 * j, 16)])

    s0w = pl.multiple_of(s0 * 128, 8)

    @pl.when(wid < 2)
    def _out_big():
        pltpu.sync_copy(acc.at[pl.ds(0, SEG_BIG * 128)],
                        xg_hbm.at[pl.ds(s0w, SEG_BIG * 128)])

    @pl.when(wid >= 2)
    def _out_small():
        pltpu.sync_copy(acc.at[pl.ds(0, SEG_SMALL * 128)],
                        xg_hbm.at[pl.ds(s0w, SEG_SMALL * 128)])


def _segment_max_sc(x, idx, q):
    mesh = plsc.VectorSubcoreMesh(core_axis_name="c", subcore_axis_name="s")
    f = functools.partial(
        pl.kernel,
        out_type=jax.ShapeDtypeStruct((S * D,), jnp.float32),
        mesh=mesh,
        scratch_types=[
            pltpu.VMEM((ACC_ROWS * D,), jnp.float32),
            pltpu.VMEM((T, D), jnp.float32),
            pltpu.VMEM((T, D), jnp.float32),
            pltpu.VMEM((T + 16,), jnp.int32),
            pltpu.VMEM((T + 16,), jnp.int32),
            pltpu.VMEM((48,), jnp.int32),
            pltpu.VMEM((16,), jnp.int32),
            pltpu.SemaphoreType.DMA,
            pltpu.SemaphoreType.DMA,
            pltpu.SemaphoreType.DMA,
            pltpu.SemaphoreType.DMA,
        ],
        compiler_params=pltpu.CompilerParams(needs_layout_passes=False),
    )(_seg_max_body)
    return f(x, idx, q).reshape(S, D)


def _mlp_body(xg_ref, w_ref, b_ref, o_ref):
    y = jnp.dot(xg_ref[...], w_ref[...], preferred_element_type=jnp.float32)
    o_ref[...] = jnp.maximum(y + b_ref[...], 0.0)


def _mlp_tc(xg, W, b):
    return pl.pallas_call(
        _mlp_body,
        grid=(10,),
        in_specs=[
            pl.BlockSpec((S // 10, D), lambda i: (i, 0)),
            pl.BlockSpec((D, DO), lambda i: (0, 0)),
            pl.BlockSpec((1, DO), lambda i: (0, 0)),
        ],
        out_specs=pl.BlockSpec((S // 10, DO), lambda i: (i, 0)),
        out_shape=jax.ShapeDtypeStruct((S, DO), jnp.float32),
    )(xg, W, b.reshape(1, DO))


def kernel(x, idx, W, b):
    w_arange = jnp.arange(NW, dtype=jnp.int32)
    seg_starts = SEG_SMALL * w_arange + 8 * jnp.minimum(w_arange, 2)
    q = jnp.searchsorted(idx, seg_starts, side="left",
                         method="compare_all").astype(jnp.int32)
    q = jnp.concatenate([q, jnp.full((16,), N, dtype=jnp.int32)])
    xg = _segment_max_sc(x, idx, q)
    return _mlp_tc(xg, W, b)
